# baseline (speedup 1.0000x reference)
import functools

import jax
import jax.numpy as jnp
from jax import lax
from jax.experimental import pallas as pl
from jax.experimental.pallas import tpu as pltpu
from jax.experimental.pallas import tpu_sc as plsc

N = 10000
E = 320000
D = 128

CHUNK = 128
NC = 2
NS = 16
NW = NC * NS
CPW = 2 * ((-(-E // (CHUNK * NW)) + 1) // 2)
CHUNKS_TOTAL = CPW * NW
BLK = 16
NBLK = CPW // BLK
E_PAD = CHUNKS_TOTAL * CHUNK
ACC_ROWS = 10240
RPT = ACC_ROWS // NS


def _sc_aggregate(x, src2d, dst2d, w2d):
    mesh = plsc.VectorSubcoreMesh(core_axis_name="c", subcore_axis_name="s")

    @functools.partial(
        pl.kernel,
        out_type=jax.ShapeDtypeStruct((NC, ACC_ROWS, D), jnp.float32),
        mesh=mesh,
        scratch_types=[
            pltpu.VMEM((2, BLK, CHUNK), jnp.int32),
            pltpu.VMEM((2, BLK, CHUNK), jnp.int32),
            pltpu.VMEM((2, BLK, CHUNK), jnp.float32),
            pltpu.VMEM((CHUNK, D), jnp.float32),
            pltpu.VMEM((CHUNK, D), jnp.float32),
            pltpu.VMEM_SHARED((ACC_ROWS, D), jnp.float32),
            pltpu.SemaphoreType.DMA,
            pltpu.SemaphoreType.DMA,
            pltpu.SemaphoreType.DMA,
            pltpu.SemaphoreType.DMA,
            pltpu.SemaphoreType.DMA,
            pltpu.SemaphoreType.DMA,
        ],
    )
    def k(x_hbm, src_hbm, dst_hbm, w_hbm, out_hbm,
          src_blk, dst_blk, w_blk, rows0, rows1, acc,
          g0, g1, s0, s1, i0, i1):
        cid = lax.axis_index("c")
        sid = lax.axis_index("s")
        wid = cid * NS + sid

        def zrow(r, carry):
            for c in range(D // 16):
                rows0[r, pl.ds(c * 16, 16)] = jnp.zeros((16,), jnp.float32)
            return carry

        lax.fori_loop(0, CHUNK, zrow, 0)
        for q in range(RPT // CHUNK):
            pltpu.sync_copy(rows0, acc.at[pl.ds(sid * RPT + q * CHUNK, CHUNK)])

        rows = (rows0, rows1)
        gsem = (g0, g1)
        ssem = (s0, s1)
        isem = (i0, i1)
        wbase = pl.multiple_of(wid * CPW, 8)

        def start_idx_load(kb, p):
            hb = pl.multiple_of(wbase + kb * BLK, 8)
            pltpu.async_copy(src_hbm.at[pl.ds(hb, BLK)], src_blk.at[p],
                             isem[p])
            pltpu.async_copy(dst_hbm.at[pl.ds(hb, BLK)], dst_blk.at[p],
                             isem[p])
            pltpu.async_copy(w_hbm.at[pl.ds(hb, BLK)], w_blk.at[p], isem[p])

        def wait_idx_load(kb, p):
            hb = pl.multiple_of(wbase + kb * BLK, 8)
            pltpu.make_async_copy(src_hbm.at[pl.ds(hb, BLK)], src_blk.at[p],
                                  isem[p]).wait()
            pltpu.make_async_copy(dst_hbm.at[pl.ds(hb, BLK)], dst_blk.at[p],
                                  isem[p]).wait()
            pltpu.make_async_copy(w_hbm.at[pl.ds(hb, BLK)], w_blk.at[p],
                                  isem[p]).wait()

        start_idx_load(0, 0)
        plsc.subcore_barrier()

        for kb in range(NBLK):
            p = kb % 2
            wait_idx_load(kb, p)
            pltpu.async_copy(x_hbm.at[src_blk.at[p, 0]], rows0, g0)
            if kb + 1 < NBLK:
                start_idx_load(kb + 1, 1 - p)

            def pair_body(t, carry):
                for b in range(2):
                    jj = t * 2 + b
                    j = kb * BLK + jj
                    rb, sb = rows[b], ssem[b]
                    rn, gn, sn = rows[1 - b], gsem[1 - b], ssem[1 - b]
                    pltpu.make_async_copy(x_hbm.at[src_blk.at[p, jj]], rb,
                                          gsem[b]).wait()

                    @pl.when(j >= 1)
                    def _():
                        pltpu.make_async_copy(rn, acc.at[dst_blk.at[p, jj]],
                                              sn).wait()

                    @pl.when(jj + 1 < BLK)
                    def _():
                        pltpu.async_copy(x_hbm.at[src_blk.at[p, jj + 1]],
                                         rn, gn)

                    def group_body(g, c2):
                        wv = w_blk[p, jj, pl.ds(g * 16, 16)]
                        for e2 in range(16):
                            ws = wv[e2]
                            row_e = g * 16 + e2
                            for c in range(D // 16):
                                sl = pl.ds(c * 16, 16)
                                rb[row_e, sl] = rb[row_e, sl] * ws
                        return c2

                    lax.fori_loop(0, CHUNK // 16, group_body, 0)
                    pltpu.async_copy(rb, acc.at[dst_blk.at[p, jj]], sb,
                                     add=True)
                return carry

            lax.fori_loop(0, BLK // 2, pair_body, 0)

        pltpu.make_async_copy(rows1, acc.at[dst_blk.at[(NBLK - 1) % 2,
                                                       BLK - 1]], s1).wait()
        plsc.subcore_barrier()
        pltpu.sync_copy(acc.at[pl.ds(sid * RPT, RPT)],
                        out_hbm.at[cid, pl.ds(sid * RPT, RPT)])

    return k(x, src2d, dst2d, w2d)


def _tc_combine(p0, p1, W):
    BM = 2000

    def body(p0_ref, p1_ref, w_ref, o_ref):
        s = p0_ref[...] + p1_ref[...]
        o_ref[...] = jnp.maximum(
            jnp.dot(s, w_ref[...], preferred_element_type=jnp.float32), 0.0)

    return pl.pallas_call(
        body,
        grid=(N // BM,),
        in_specs=[
            pl.BlockSpec((BM, D), lambda i: (i, 0)),
            pl.BlockSpec((BM, D), lambda i: (i, 0)),
            pl.BlockSpec((D, D), lambda i: (0, 0)),
        ],
        out_specs=pl.BlockSpec((BM, D), lambda i: (i, 0)),
        out_shape=jax.ShapeDtypeStruct((N, D), jnp.float32),
    )(p0, p1, W)


@jax.jit
def kernel(x, edge_index, edge_weight, W):
    pad = E_PAD - E
    src = jnp.concatenate([edge_index[1], jnp.zeros((pad,), jnp.int32)])
    dst = jnp.concatenate([edge_index[0], jnp.zeros((pad,), jnp.int32)])
    w = jnp.concatenate([edge_weight, jnp.zeros((pad,), jnp.float32)])
    src2d = src.reshape(CHUNKS_TOTAL, CHUNK)
    dst2d = dst.reshape(CHUNKS_TOTAL, CHUNK)
    w2d = w.reshape(CHUNKS_TOTAL, CHUNK)
    partials = _sc_aggregate(x, src2d, dst2d, w2d)
    return _tc_combine(partials[0, :N], partials[1, :N], W)


# speedup vs baseline: 1.0574x; 1.0574x over previous
import functools

import jax
import jax.numpy as jnp
from jax import lax
from jax.experimental import pallas as pl
from jax.experimental.pallas import tpu as pltpu
from jax.experimental.pallas import tpu_sc as plsc

N = 10000
E = 320000
D = 128

CHUNK = 128
NC = 2
NS = 16
NW = NC * NS
CPW = 2 * ((-(-E // (CHUNK * NW)) + 1) // 2)
CHUNKS_TOTAL = CPW * NW
BLK = 16
NBLK = CPW // BLK
E_PAD = CHUNKS_TOTAL * CHUNK
ACC_ROWS = 10240
RPT = ACC_ROWS // NS


def _sc_aggregate(x2, srcc, dst2d, w2d):
    mesh = plsc.VectorSubcoreMesh(core_axis_name="c", subcore_axis_name="s")

    @functools.partial(
        pl.kernel,
        out_type=jax.ShapeDtypeStruct((NC, ACC_ROWS, D), jnp.float32),
        mesh=mesh,
        scratch_types=[
            pltpu.VMEM((2, BLK, CHUNK), jnp.int32),
            pltpu.VMEM((2, BLK, CHUNK), jnp.int32),
            pltpu.VMEM((2, BLK, CHUNK), jnp.float32),
            pltpu.VMEM((CHUNK, D), jnp.float32),
            pltpu.VMEM((CHUNK, D), jnp.float32),
            pltpu.VMEM_SHARED((ACC_ROWS, D), jnp.float32),
            pltpu.SemaphoreType.DMA,
            pltpu.SemaphoreType.DMA,
            pltpu.SemaphoreType.DMA,
            pltpu.SemaphoreType.DMA,
            pltpu.SemaphoreType.DMA,
            pltpu.SemaphoreType.DMA,
        ],
    )
    def k(x2_hbm, srcc_hbm, dst_hbm, w_hbm, out_hbm,
          src_blk, dst_blk, w_blk, rows0, rows1, acc,
          g0, g1, s0, s1, i0, i1):
        cid = lax.axis_index("c")
        sid = lax.axis_index("s")
        wid = cid * NS + sid

        def zrow(r, carry):
            for c in range(D // 16):
                rows0[r, pl.ds(c * 16, 16)] = jnp.zeros((16,), jnp.float32)
            return carry

        lax.fori_loop(0, CHUNK, zrow, 0)
        for q in range(RPT // CHUNK):
            pltpu.sync_copy(rows0, acc.at[pl.ds(sid * RPT + q * CHUNK, CHUNK)])

        rows = (rows0, rows1)
        gsem = (g0, g1)
        ssem = (s0, s1)
        isem = (i0, i1)
        wbase = pl.multiple_of(wid * CPW, 8)

        def start_idx_load(kb, p):
            hb = pl.multiple_of(wbase + kb * BLK, 8)
            pltpu.async_copy(srcc_hbm.at[cid, pl.ds(hb, BLK)], src_blk.at[p],
                             isem[p])
            pltpu.async_copy(dst_hbm.at[pl.ds(hb, BLK)], dst_blk.at[p],
                             isem[p])
            pltpu.async_copy(w_hbm.at[pl.ds(hb, BLK)], w_blk.at[p], isem[p])

        def start_gather(idx_ref, dst_buf, sem):
            pltpu.async_copy(x2_hbm.at[idx_ref], dst_buf, sem)

        def wait_idx_load(kb, p):
            hb = pl.multiple_of(wbase + kb * BLK, 8)
            pltpu.make_async_copy(srcc_hbm.at[cid, pl.ds(hb, BLK)],
                                  src_blk.at[p], isem[p]).wait()
            pltpu.make_async_copy(dst_hbm.at[pl.ds(hb, BLK)], dst_blk.at[p],
                                  isem[p]).wait()
            pltpu.make_async_copy(w_hbm.at[pl.ds(hb, BLK)], w_blk.at[p],
                                  isem[p]).wait()

        start_idx_load(0, 0)
        plsc.subcore_barrier()

        for kb in range(NBLK):
            p = kb % 2
            wait_idx_load(kb, p)
            start_gather(src_blk.at[p, 0], rows0, g0)
            if kb + 1 < NBLK:
                start_idx_load(kb + 1, 1 - p)

            def pair_body(t, carry):
                for b in range(2):
                    jj = t * 2 + b
                    j = kb * BLK + jj
                    rb, sb = rows[b], ssem[b]
                    rn, gn, sn = rows[1 - b], gsem[1 - b], ssem[1 - b]
                    pltpu.make_async_copy(x2_hbm.at[src_blk.at[p, jj]], rb,
                                          gsem[b]).wait()

                    @pl.when(j >= 1)
                    def _():
                        pltpu.make_async_copy(rn, acc.at[dst_blk.at[p, jj]],
                                              sn).wait()

                    @pl.when(jj + 1 < BLK)
                    def _():
                        start_gather(src_blk.at[p, jj + 1], rn, gn)

                    def group_body(g, c2):
                        wv = w_blk[p, jj, pl.ds(g * 16, 16)]
                        for e2 in range(16):
                            ws = wv[e2]
                            row_e = g * 16 + e2
                            for c in range(D // 16):
                                sl = pl.ds(c * 16, 16)
                                rb[row_e, sl] = rb[row_e, sl] * ws
                        return c2

                    lax.fori_loop(0, CHUNK // 16, group_body, 0)
                    pltpu.async_copy(rb, acc.at[dst_blk.at[p, jj]], sb,
                                     add=True)
                return carry

            lax.fori_loop(0, BLK // 2, pair_body, 0)

        pltpu.make_async_copy(rows1, acc.at[dst_blk.at[(NBLK - 1) % 2,
                                                       BLK - 1]], s1).wait()
        plsc.subcore_barrier()
        pltpu.sync_copy(acc.at[pl.ds(sid * RPT, RPT)],
                        out_hbm.at[cid, pl.ds(sid * RPT, RPT)])

    return k(x2, srcc, dst2d, w2d)


def _tc_combine(p0, p1, W):
    BM = 2000

    def body(p0_ref, p1_ref, w_ref, o_ref):
        s = p0_ref[...] + p1_ref[...]
        o_ref[...] = jnp.maximum(
            jnp.dot(s, w_ref[...], preferred_element_type=jnp.float32), 0.0)

    return pl.pallas_call(
        body,
        grid=(N // BM,),
        in_specs=[
            pl.BlockSpec((BM, D), lambda i: (i, 0)),
            pl.BlockSpec((BM, D), lambda i: (i, 0)),
            pl.BlockSpec((D, D), lambda i: (0, 0)),
        ],
        out_specs=pl.BlockSpec((BM, D), lambda i: (i, 0)),
        out_shape=jax.ShapeDtypeStruct((N, D), jnp.float32),
    )(p0, p1, W)


@jax.jit
def kernel(x, edge_index, edge_weight, W):
    pad = E_PAD - E
    src = jnp.concatenate([edge_index[1], jnp.zeros((pad,), jnp.int32)])
    dst = jnp.concatenate([edge_index[0], jnp.zeros((pad,), jnp.int32)])
    w = jnp.concatenate([edge_weight, jnp.zeros((pad,), jnp.float32)])
    src2d = src.reshape(CHUNKS_TOTAL, CHUNK)
    dst2d = dst.reshape(CHUNKS_TOTAL, CHUNK)
    w2d = w.reshape(CHUNKS_TOTAL, CHUNK)
    x2 = jnp.concatenate([x, x[::-1]])
    srcc = jnp.stack([src2d, (2 * N - 1) - src2d])
    partials = _sc_aggregate(x2, srcc, dst2d, w2d)
    return _tc_combine(partials[0, :N], partials[1, :N], W)
